# Initial kernel scaffold; baseline (speedup 1.0000x reference)
import jax
import jax.numpy as jnp
from jax.experimental import pallas as pl


def kernel(x, edge_index, W0, b0, W1, b1, W2, b2, fcW, fcb):
    raise NotImplementedError("write your pallas kernel here")


# speedup vs baseline: 9.5836x; 9.5836x over previous
import functools

import jax
import jax.numpy as jnp
from jax import lax
from jax.experimental import pallas as pl
from jax.experimental.pallas import tpu as pltpu
from jax.experimental.pallas import tpu_sc as plsc

N = 10000
E = 640000
D = 128

NC = 2
NS = 16
NW = NC * NS
EPW = E // NW
G = 80
CH = EPW // G
CB = 25
NB = CH // CB
NP = 10240
RPT = NP // NS

_mesh = plsc.VectorSubcoreMesh(core_axis_name="c", subcore_axis_name="s")


@functools.partial(
    pl.kernel,
    mesh=_mesh,
    out_type=jax.ShapeDtypeStruct((NC, NP, D), jnp.float32),
    scratch_types=[
        pltpu.VMEM((CB, G), jnp.int32),
        pltpu.VMEM((CB, G), jnp.int32),
        pltpu.VMEM((G, D), jnp.float32),
        pltpu.VMEM_SHARED((NP, D), jnp.float32),
        pltpu.SemaphoreType.DMA,
    ],
)
def _aggregate(h_hbm, src_hbm, dst_hbm, zeros_hbm, out_hbm,
               src_v, dst_v, msg_v, acc, sem):
    cid = lax.axis_index("c")
    sid = lax.axis_index("s")
    wid = sid * NC + cid

    row0 = pl.multiple_of(sid * RPT, 8)
    pltpu.sync_copy(zeros_hbm, acc.at[pl.ds(row0, RPT)])
    plsc.subcore_barrier()

    def outer(bi, carry):
        pltpu.sync_copy(src_hbm.at[wid, bi], src_v)
        pltpu.sync_copy(dst_hbm.at[wid, bi], dst_v)

        def body(ci, c):
            pltpu.async_copy(h_hbm.at[src_v.at[ci]], msg_v, sem).wait()
            pltpu.sync_copy(msg_v, acc.at[dst_v.at[ci]], add=True)
            return c

        return lax.fori_loop(0, CB, body, carry, unroll=False)

    lax.fori_loop(0, NB, outer, 0, unroll=False)

    plsc.subcore_barrier()
    pltpu.sync_copy(acc.at[pl.ds(row0, RPT)],
                    out_hbm.at[cid, pl.ds(row0, RPT)])


BLK = 400


def _matmul_body(p_ref, w_ref, b_ref, o_ref):
    h = p_ref[0] + p_ref[1]
    o_ref[...] = (
        jnp.dot(h, w_ref[...], preferred_element_type=jnp.float32) + b_ref[...]
    )


def _combine_matmul(p, w, b):
    return pl.pallas_call(
        _matmul_body,
        grid=(N // BLK,),
        in_specs=[
            pl.BlockSpec((2, BLK, D), lambda i: (0, i, 0)),
            pl.BlockSpec((D, D), lambda i: (0, 0)),
            pl.BlockSpec((1, D), lambda i: (0, 0)),
        ],
        out_specs=pl.BlockSpec((BLK, D), lambda i: (i, 0)),
        out_shape=jax.ShapeDtypeStruct((N, D), jnp.float32),
    )(p, w, b)


def _final_body(p_ref, w2_ref, fcw_ref, b_ref, o_ref):
    h = p_ref[0] + p_ref[1]
    wc = jnp.dot(w2_ref[...], fcw_ref[...], preferred_element_type=jnp.float32)
    o_ref[...] = jnp.dot(h, wc, preferred_element_type=jnp.float32) + b_ref[...]


def _final_matmul(p, w2, fcw, b2, fcb):
    bc = jnp.concatenate([b2[None, :], fcb[None, :]], axis=0)

    def body(p_ref, w2_ref, fcw_ref, bc_ref, o_ref):
        h = p_ref[0] + p_ref[1]
        wc = jnp.dot(w2_ref[...], fcw_ref[...],
                     preferred_element_type=jnp.float32)
        bias = (
            jnp.dot(bc_ref[0:1, :], fcw_ref[...],
                    preferred_element_type=jnp.float32)
            + bc_ref[1:2, :]
        )
        o_ref[...] = (
            jnp.dot(h, wc, preferred_element_type=jnp.float32) + bias
        )

    return pl.pallas_call(
        body,
        grid=(N // BLK,),
        in_specs=[
            pl.BlockSpec((2, BLK, D), lambda i: (0, i, 0)),
            pl.BlockSpec((D, D), lambda i: (0, 0)),
            pl.BlockSpec((D, D), lambda i: (0, 0)),
            pl.BlockSpec((2, D), lambda i: (0, 0)),
        ],
        out_specs=pl.BlockSpec((BLK, D), lambda i: (i, 0)),
        out_shape=jax.ShapeDtypeStruct((N, D), jnp.float32),
    )(p, w2, fcw, bc)


def kernel(x, edge_index, W0, b0, W1, b1, W2, b2, fcW, fcb):
    src = edge_index[0].reshape(NW, NB, CB, G)
    dst = edge_index[1].reshape(NW, NB, CB, G)
    zeros = jnp.zeros((RPT, D), jnp.float32)

    p = _aggregate(x, src, dst, zeros)
    h = _combine_matmul(p, W0, b0[None, :])
    p = _aggregate(h, src, dst, zeros)
    h = _combine_matmul(p, W1, b1[None, :])
    p = _aggregate(h, src, dst, zeros)
    return _final_matmul(p, W2, fcW, b2, fcb)


# speedup vs baseline: 13.9025x; 1.4507x over previous
import functools

import jax
import jax.numpy as jnp
from jax import lax
from jax.experimental import pallas as pl
from jax.experimental.pallas import tpu as pltpu
from jax.experimental.pallas import tpu_sc as plsc

N = 10000
E = 640000
D = 128

NC = 2
NS = 16
NW = NC * NS
EPW = E // NW
G = 80
CH = EPW // G
CB = 10
NB = CH // CB
NP = 10240
RPT = NP // NS

_mesh = plsc.VectorSubcoreMesh(core_axis_name="c", subcore_axis_name="s")


@functools.partial(
    pl.kernel,
    mesh=_mesh,
    out_type=jax.ShapeDtypeStruct((NC, NP, D), jnp.float32),
    scratch_types=[
        pltpu.VMEM((CB, G), jnp.int32),
        pltpu.VMEM((CB, G), jnp.int32),
        pltpu.VMEM((G, D), jnp.float32),
        pltpu.VMEM((G, D), jnp.float32),
        pltpu.VMEM_SHARED((NP, D), jnp.float32),
        pltpu.SemaphoreType.DMA,
        pltpu.SemaphoreType.DMA,
    ],
)
def _aggregate(h_hbm, src_hbm, dst_hbm, zeros_hbm, out_hbm,
               src_v, dst_v, msg0, msg1, acc, sem0, sem1):
    cid = lax.axis_index("c")
    sid = lax.axis_index("s")
    wid = sid * NC + cid

    row0 = pl.multiple_of(sid * RPT, 8)
    pltpu.sync_copy(zeros_hbm, acc.at[pl.ds(row0, RPT)])
    plsc.subcore_barrier()

    def outer(bi, carry):
        pltpu.sync_copy(src_hbm.at[wid, bi], src_v)
        pltpu.sync_copy(dst_hbm.at[wid, bi], dst_v)

        pltpu.async_copy(h_hbm.at[src_v.at[0]], msg0, sem0)

        def pair(pi, c):
            ci = 2 * pi
            pltpu.async_copy(h_hbm.at[src_v.at[ci + 1]], msg1, sem1)
            pltpu.make_async_copy(h_hbm.at[src_v.at[ci]], msg0, sem0).wait()
            pltpu.sync_copy(msg0, acc.at[dst_v.at[ci]], add=True)

            @pl.when(pi + 1 < CB // 2)
            def _():
                pltpu.async_copy(h_hbm.at[src_v.at[ci + 2]], msg0, sem0)

            pltpu.make_async_copy(h_hbm.at[src_v.at[ci + 1]], msg1, sem1).wait()
            pltpu.sync_copy(msg1, acc.at[dst_v.at[ci + 1]], add=True)
            return c

        return lax.fori_loop(0, CB // 2, pair, carry, unroll=False)

    lax.fori_loop(0, NB, outer, 0, unroll=False)

    plsc.subcore_barrier()
    pltpu.sync_copy(acc.at[pl.ds(row0, RPT)],
                    out_hbm.at[cid, pl.ds(row0, RPT)])


BLK = 400


def _matmul_body(p_ref, w_ref, b_ref, o_ref):
    h = p_ref[0] + p_ref[1]
    o_ref[...] = (
        jnp.dot(h, w_ref[...], preferred_element_type=jnp.float32) + b_ref[...]
    )


def _combine_matmul(p, w, b):
    return pl.pallas_call(
        _matmul_body,
        grid=(N // BLK,),
        in_specs=[
            pl.BlockSpec((2, BLK, D), lambda i: (0, i, 0)),
            pl.BlockSpec((D, D), lambda i: (0, 0)),
            pl.BlockSpec((1, D), lambda i: (0, 0)),
        ],
        out_specs=pl.BlockSpec((BLK, D), lambda i: (i, 0)),
        out_shape=jax.ShapeDtypeStruct((N, D), jnp.float32),
    )(p, w, b)


def _final_body(p_ref, w2_ref, fcw_ref, b_ref, o_ref):
    h = p_ref[0] + p_ref[1]
    wc = jnp.dot(w2_ref[...], fcw_ref[...], preferred_element_type=jnp.float32)
    o_ref[...] = jnp.dot(h, wc, preferred_element_type=jnp.float32) + b_ref[...]


def _final_matmul(p, w2, fcw, b2, fcb):
    bc = jnp.concatenate([b2[None, :], fcb[None, :]], axis=0)

    def body(p_ref, w2_ref, fcw_ref, bc_ref, o_ref):
        h = p_ref[0] + p_ref[1]
        wc = jnp.dot(w2_ref[...], fcw_ref[...],
                     preferred_element_type=jnp.float32)
        bias = (
            jnp.dot(bc_ref[0:1, :], fcw_ref[...],
                    preferred_element_type=jnp.float32)
            + bc_ref[1:2, :]
        )
        o_ref[...] = (
            jnp.dot(h, wc, preferred_element_type=jnp.float32) + bias
        )

    return pl.pallas_call(
        body,
        grid=(N // BLK,),
        in_specs=[
            pl.BlockSpec((2, BLK, D), lambda i: (0, i, 0)),
            pl.BlockSpec((D, D), lambda i: (0, 0)),
            pl.BlockSpec((D, D), lambda i: (0, 0)),
            pl.BlockSpec((2, D), lambda i: (0, 0)),
        ],
        out_specs=pl.BlockSpec((BLK, D), lambda i: (i, 0)),
        out_shape=jax.ShapeDtypeStruct((N, D), jnp.float32),
    )(p, w2, fcw, bc)


def kernel(x, edge_index, W0, b0, W1, b1, W2, b2, fcW, fcb):
    src = edge_index[0].reshape(NW, NB, CB, G)
    dst = edge_index[1].reshape(NW, NB, CB, G)
    zeros = jnp.zeros((RPT, D), jnp.float32)

    p = _aggregate(x, src, dst, zeros)
    h = _combine_matmul(p, W0, b0[None, :])
    p = _aggregate(h, src, dst, zeros)
    h = _combine_matmul(p, W1, b1[None, :])
    p = _aggregate(h, src, dst, zeros)
    return _final_matmul(p, W2, fcW, b2, fcb)


# speedup vs baseline: 15.9666x; 1.1485x over previous
import functools

import jax
import jax.numpy as jnp
from jax import lax
from jax.experimental import pallas as pl
from jax.experimental.pallas import tpu as pltpu
from jax.experimental.pallas import tpu_sc as plsc

N = 10000
E = 640000
D = 128

NC = 2
NS = 16
NW = NC * NS
EPW = E // NW
G = 80
CH = EPW // G
CB = 50
NB = CH // CB
NP = 10240
RPT = NP // NS

_mesh = plsc.VectorSubcoreMesh(core_axis_name="c", subcore_axis_name="s")


@functools.partial(
    pl.kernel,
    mesh=_mesh,
    out_type=jax.ShapeDtypeStruct((NC, NP, D), jnp.float32),
    scratch_types=[
        pltpu.VMEM((CB, G), jnp.int32),
        pltpu.VMEM((CB, G), jnp.int32),
        pltpu.VMEM((G, D), jnp.float32),
        pltpu.VMEM((G, D), jnp.float32),
        pltpu.VMEM_SHARED((NP, D), jnp.float32),
        pltpu.SemaphoreType.DMA,
        pltpu.SemaphoreType.DMA,
    ],
)
def _aggregate(h_hbm, src_hbm, dst_hbm, zeros_hbm, out_hbm,
               src_v, dst_v, msg0, msg1, acc, sem0, sem1):
    cid = lax.axis_index("c")
    sid = lax.axis_index("s")
    wid = sid * NC + cid

    row0 = pl.multiple_of(sid * RPT, 8)
    pltpu.sync_copy(zeros_hbm, acc.at[pl.ds(row0, RPT)])
    plsc.subcore_barrier()

    def outer(bi, carry):
        pltpu.sync_copy(src_hbm.at[wid, bi], src_v)
        pltpu.sync_copy(dst_hbm.at[wid, bi], dst_v)

        pltpu.async_copy(h_hbm.at[src_v.at[0]], msg0, sem0)

        def pair(pi, c):
            ci = 2 * pi
            pltpu.async_copy(h_hbm.at[src_v.at[ci + 1]], msg1, sem1)
            pltpu.make_async_copy(h_hbm.at[src_v.at[ci]], msg0, sem0).wait()
            pltpu.sync_copy(msg0, acc.at[dst_v.at[ci]], add=True)

            @pl.when(pi + 1 < CB // 2)
            def _():
                pltpu.async_copy(h_hbm.at[src_v.at[ci + 2]], msg0, sem0)

            pltpu.make_async_copy(h_hbm.at[src_v.at[ci + 1]], msg1, sem1).wait()
            pltpu.sync_copy(msg1, acc.at[dst_v.at[ci + 1]], add=True)
            return c

        return lax.fori_loop(0, CB // 2, pair, carry, unroll=False)

    lax.fori_loop(0, NB, outer, 0, unroll=False)

    plsc.subcore_barrier()
    pltpu.sync_copy(acc.at[pl.ds(row0, RPT)],
                    out_hbm.at[cid, pl.ds(row0, RPT)])


BLK = 400


def _matmul_body(p_ref, w_ref, b_ref, o_ref):
    h = p_ref[0] + p_ref[1]
    o_ref[...] = (
        jnp.dot(h, w_ref[...], preferred_element_type=jnp.float32) + b_ref[...]
    )


def _combine_matmul(p, w, b):
    return pl.pallas_call(
        _matmul_body,
        grid=(N // BLK,),
        in_specs=[
            pl.BlockSpec((2, BLK, D), lambda i: (0, i, 0)),
            pl.BlockSpec((D, D), lambda i: (0, 0)),
            pl.BlockSpec((1, D), lambda i: (0, 0)),
        ],
        out_specs=pl.BlockSpec((BLK, D), lambda i: (i, 0)),
        out_shape=jax.ShapeDtypeStruct((N, D), jnp.float32),
    )(p, w, b)


def _final_body(p_ref, w2_ref, fcw_ref, b_ref, o_ref):
    h = p_ref[0] + p_ref[1]
    wc = jnp.dot(w2_ref[...], fcw_ref[...], preferred_element_type=jnp.float32)
    o_ref[...] = jnp.dot(h, wc, preferred_element_type=jnp.float32) + b_ref[...]


def _final_matmul(p, w2, fcw, b2, fcb):
    bc = jnp.concatenate([b2[None, :], fcb[None, :]], axis=0)

    def body(p_ref, w2_ref, fcw_ref, bc_ref, o_ref):
        h = p_ref[0] + p_ref[1]
        wc = jnp.dot(w2_ref[...], fcw_ref[...],
                     preferred_element_type=jnp.float32)
        bias = (
            jnp.dot(bc_ref[0:1, :], fcw_ref[...],
                    preferred_element_type=jnp.float32)
            + bc_ref[1:2, :]
        )
        o_ref[...] = (
            jnp.dot(h, wc, preferred_element_type=jnp.float32) + bias
        )

    return pl.pallas_call(
        body,
        grid=(N // BLK,),
        in_specs=[
            pl.BlockSpec((2, BLK, D), lambda i: (0, i, 0)),
            pl.BlockSpec((D, D), lambda i: (0, 0)),
            pl.BlockSpec((D, D), lambda i: (0, 0)),
            pl.BlockSpec((2, D), lambda i: (0, 0)),
        ],
        out_specs=pl.BlockSpec((BLK, D), lambda i: (i, 0)),
        out_shape=jax.ShapeDtypeStruct((N, D), jnp.float32),
    )(p, w2, fcw, bc)


def kernel(x, edge_index, W0, b0, W1, b1, W2, b2, fcW, fcb):
    src = edge_index[0].reshape(NW, NB, CB, G)
    dst = edge_index[1].reshape(NW, NB, CB, G)
    zeros = jnp.zeros((RPT, D), jnp.float32)

    p = _aggregate(x, src, dst, zeros)
    h = _combine_matmul(p, W0, b0[None, :])
    p = _aggregate(h, src, dst, zeros)
    h = _combine_matmul(p, W1, b1[None, :])
    p = _aggregate(h, src, dst, zeros)
    return _final_matmul(p, W2, fcW, b2, fcb)
